# Initial kernel scaffold; baseline (speedup 1.0000x reference)
import jax
import jax.numpy as jnp
from jax.experimental import pallas as pl


def kernel(user_emb, item_emb, edge_row, edge_col):
    raise NotImplementedError("write your pallas kernel here")


# speedup vs baseline: 5.6129x; 5.6129x over previous
import functools

import jax
import jax.numpy as jnp
from jax import lax
from jax.experimental import pallas as pl
from jax.experimental.pallas import tpu as pltpu
from jax.experimental.pallas import tpu_sc as plsc

U_N = 50000
I_N = 25000
E_N = 800000
HALF = 32
N_TILES = 16

EP = 819200
ROWS = EP // 128
RPT = ROWS // N_TILES
CHUNK = 8
N_CHUNKS = RPT // CHUNK

U_PAD = 50048
I_PAD = 25088
U_ACC = 51200
I_ACC = 26624
DEG_PAD = 50048
DEG_ACC = 51200

_MESH = plsc.VectorSubcoreMesh(core_axis_name="c", subcore_axis_name="s")


def _make_spmm(n_dst_pad, n_acc):
    zrows = n_acc // N_TILES
    orows = n_dst_pad // N_TILES

    @functools.partial(
        pl.kernel,
        out_type=jax.ShapeDtypeStruct((2 * n_dst_pad, HALF), jnp.float32),
        mesh=_MESH,
        scratch_types=[
            pltpu.VMEM((2 * CHUNK, 128), jnp.int32),
            pltpu.VMEM((2 * CHUNK, 128), jnp.int32),
            pltpu.VMEM((256, HALF), jnp.float32),
            pltpu.VMEM_SHARED((n_acc, HALF), jnp.float32),
            pltpu.SemaphoreType.DMA,
            pltpu.SemaphoreType.DMA,
            pltpu.SemaphoreType.DMA,
        ],
        compiler_params=pltpu.CompilerParams(use_tc_tiling_on_sc=False),
    )
    def spmm(src, sidx, didx, zeros, out, vsi, vdi, vbuf, acc,
             gsem, sisem, disem):
        c = lax.axis_index("c")
        s = lax.axis_index("s")
        t0 = s * RPT

        pltpu.sync_copy(zeros.at[pl.ds(0, zrows)],
                        acc.at[pl.ds(s * zrows, zrows)])
        pltpu.async_copy(sidx.at[pl.ds(c * ROWS + t0, CHUNK)],
                         vsi.at[pl.ds(0, CHUNK)], sisem)
        pltpu.async_copy(didx.at[pl.ds(t0, CHUNK)],
                         vdi.at[pl.ds(0, CHUNK)], disem)
        plsc.subcore_barrier()

        def chunk_body(k, carry):
            par = lax.rem(k, 2)
            base = par * CHUNK
            pltpu.make_async_copy(
                sidx.at[pl.ds(c * ROWS + t0, CHUNK)],
                vsi.at[pl.ds(base, CHUNK)], sisem).wait()
            pltpu.make_async_copy(
                didx.at[pl.ds(t0, CHUNK)],
                vdi.at[pl.ds(base, CHUNK)], disem).wait()

            @pl.when(k < N_CHUNKS - 1)
            def _():
                nb = (1 - par) * CHUNK
                off = (k + 1) * CHUNK
                pltpu.async_copy(sidx.at[pl.ds(c * ROWS + t0 + off, CHUNK)],
                                 vsi.at[pl.ds(nb, CHUNK)], sisem)
                pltpu.async_copy(didx.at[pl.ds(t0 + off, CHUNK)],
                                 vdi.at[pl.ds(nb, CHUNK)], disem)

            descs = [pltpu.async_copy(src.at[vsi.at[base]],
                                      vbuf.at[pl.ds(0, 128)], gsem)]
            for j in range(CHUNK):
                bpar = j % 2
                if j + 1 < CHUNK:
                    descs.append(
                        pltpu.async_copy(src.at[vsi.at[base + j + 1]],
                                         vbuf.at[pl.ds((1 - bpar) * 128, 128)],
                                         gsem))
                descs[j].wait()
                pltpu.sync_copy(vbuf.at[pl.ds(bpar * 128, 128)],
                                acc.at[vdi.at[base + j]], add=True)
            return carry

        lax.fori_loop(0, N_CHUNKS, chunk_body, 0)
        plsc.subcore_barrier()

        pltpu.sync_copy(acc.at[pl.ds(s * orows, orows)],
                        out.at[pl.ds(c * n_dst_pad + s * orows, orows)])

    return spmm


_spmm_u2i = _make_spmm(I_PAD, I_ACC)
_spmm_i2u = _make_spmm(U_PAD, U_ACC)


@functools.partial(
    pl.kernel,
    out_type=jax.ShapeDtypeStruct((2 * DEG_PAD, 1), jnp.float32),
    mesh=_MESH,
    scratch_types=[
        pltpu.VMEM((RPT, 128), jnp.int32),
        pltpu.VMEM((128, 1), jnp.float32),
        pltpu.VMEM_SHARED((DEG_ACC, 1), jnp.float32),
        pltpu.SemaphoreType.DMA,
    ],
    compiler_params=pltpu.CompilerParams(use_tc_tiling_on_sc=False),
)
def _degrees(didx, zeros1, ones, out, vdi, ones_v, acc, sem):
    c = lax.axis_index("c")
    s = lax.axis_index("s")
    t0 = s * RPT
    zrows = DEG_ACC // N_TILES
    orows = DEG_PAD // N_TILES

    pltpu.sync_copy(ones, ones_v)
    pltpu.sync_copy(zeros1.at[pl.ds(0, zrows)],
                    acc.at[pl.ds(s * zrows, zrows)])
    pltpu.sync_copy(didx.at[pl.ds(c * ROWS + t0, RPT)], vdi)
    plsc.subcore_barrier()

    def chunk_body(k, carry):
        base = k * CHUNK
        descs = []
        for j in range(CHUNK):
            descs.append(
                pltpu.async_copy(ones_v, acc.at[vdi.at[base + j]], sem,
                                 add=True))
        for d in descs:
            d.wait()
        return carry

    lax.fori_loop(0, N_CHUNKS, chunk_body, 0)
    plsc.subcore_barrier()
    pltpu.sync_copy(acc.at[pl.ds(s * orows, orows)],
                    out.at[pl.ds(c * DEG_PAD + s * orows, orows)])


def _half(x, n, n_pad):
    q = x.reshape(n, 2, HALF).transpose(1, 0, 2)
    q = jnp.pad(q, ((0, 0), (0, n_pad - n), (0, 0)))
    return q.reshape(2 * n_pad, HALF)


def _unhalf(x, n, n_pad):
    return x.reshape(2, n_pad, HALF)[:, :n].transpose(1, 0, 2).reshape(n, 64)


def _mult(vals, n, n_pad):
    return jnp.tile(jnp.pad(vals, (0, n_pad - n)), 2)[:, None]


def kernel(user_emb, item_emb, edge_row, edge_col):
    pad = EP - E_N
    pr = jnp.concatenate([edge_row, jnp.full((pad,), U_N, jnp.int32)])
    pc = jnp.concatenate([edge_col, jnp.full((pad,), I_N, jnp.int32)])
    dsti = jnp.concatenate([pr, pc]).reshape(2 * ROWS, 128)
    row_dst = dsti[:ROWS]
    col_dst = dsti[ROWS:]

    prs = jnp.concatenate([edge_row, jnp.zeros((pad,), jnp.int32)])
    pcs = jnp.concatenate([edge_col, jnp.zeros((pad,), jnp.int32)])
    usrc = jnp.concatenate([prs, prs + U_PAD]).reshape(2 * ROWS, 128)
    isrc = jnp.concatenate([pcs, pcs + I_PAD]).reshape(2 * ROWS, 128)

    zeros_sp = jnp.zeros((U_ACC // N_TILES, HALF), jnp.float32)
    zeros_1 = jnp.zeros((DEG_ACC // N_TILES, 1), jnp.float32)
    ones_128 = jnp.ones((128, 1), jnp.float32)

    deg = _degrees(dsti, zeros_1, ones_128)[:, 0]
    u_deg = jnp.where(deg[:U_N] == 0, 1.0, deg[:U_N])
    i_deg_raw = deg[DEG_PAD:DEG_PAD + I_N]
    i_deg = jnp.where(i_deg_raw == 0, 1.0, i_deg_raw)
    m_dus = _mult(u_deg ** -0.5, U_N, U_PAD)
    m_dis = _mult(i_deg ** -0.5, I_N, I_PAD)
    m_dui = _mult(1.0 / u_deg, U_N, U_PAD)
    m_dii = _mult(1.0 / i_deg, I_N, I_PAD)

    def u2i(x):
        return _spmm_u2i(x, usrc, col_dst, zeros_sp)

    def i2u(x):
        return _spmm_i2u(x, isrc, row_dst, zeros_sp)

    ue = _half(user_emb, U_N, U_PAD)
    ie = _half(item_emb, I_N, I_PAD)
    sum_u = ue
    sum_i = ie

    for _ in range(2):
        t = u2i(ue * m_dus)
        t = i2u(t * m_dii)
        ue = t * m_dus
        sum_u = sum_u + ue
        t = i2u(ie * m_dis)
        t = u2i(t * m_dui)
        ie = t * m_dis
        sum_i = sum_i + ie

    user_out = _unhalf(sum_u / 3.0, U_N, U_PAD)
    item_out = _unhalf(sum_i / 3.0, I_N, I_PAD)
    return (user_out, item_out)


# speedup vs baseline: 6.2228x; 1.1086x over previous
import functools

import jax
import jax.numpy as jnp
from jax import lax
from jax.experimental import pallas as pl
from jax.experimental.pallas import tpu as pltpu
from jax.experimental.pallas import tpu_sc as plsc

U_N = 50000
I_N = 25000
E_N = 800000
HALF = 32
N_TILES = 16

EP = 819200
ROWS = EP // 128
RPT = ROWS // N_TILES
CHUNK = 8
N_CHUNKS = RPT // CHUNK

U_PAD = 50048
I_PAD = 25088
U_ACC = 51200
I_ACC = 26624
DEG_PAD = 50048
DEG_ACC = 51200

_MESH = plsc.VectorSubcoreMesh(core_axis_name="c", subcore_axis_name="s")


def _make_spmm(n_dst_pad, n_acc):
    zrows = n_acc // N_TILES
    orows = n_dst_pad // N_TILES
    NBUF = 4
    IDXB = 8
    NBLK = RPT // IDXB

    @functools.partial(
        pl.kernel,
        out_type=jax.ShapeDtypeStruct((2 * n_dst_pad, HALF), jnp.float32),
        mesh=_MESH,
        scratch_types=[
            pltpu.VMEM((3 * IDXB, 128), jnp.int32),
            pltpu.VMEM((3 * IDXB, 128), jnp.int32),
            pltpu.VMEM((NBUF * 128, HALF), jnp.float32),
            pltpu.VMEM_SHARED((n_acc, HALF), jnp.float32),
            pltpu.SemaphoreType.DMA,
            pltpu.SemaphoreType.DMA,
            pltpu.SemaphoreType.DMA,
            pltpu.SemaphoreType.DMA,
        ],
        compiler_params=pltpu.CompilerParams(use_tc_tiling_on_sc=False),
    )
    def spmm(src, sidx, didx, zeros, out, vsi, vdi, vbuf, acc,
             gsem, ssem, sisem, disem):
        c = lax.axis_index("c")
        s = lax.axis_index("s")
        t0 = s * RPT

        def idx_issue(blk, slot):
            pltpu.async_copy(sidx.at[pl.ds(c * ROWS + t0 + blk * IDXB, IDXB)],
                             vsi.at[pl.ds(slot * IDXB, IDXB)], sisem)
            pltpu.async_copy(didx.at[pl.ds(t0 + blk * IDXB, IDXB)],
                             vdi.at[pl.ds(slot * IDXB, IDXB)], disem)

        def idx_wait():
            pltpu.make_async_copy(sidx.at[pl.ds(t0, IDXB)],
                                  vsi.at[pl.ds(0, IDXB)], sisem).wait()
            pltpu.make_async_copy(didx.at[pl.ds(t0, IDXB)],
                                  vdi.at[pl.ds(0, IDXB)], disem).wait()

        def gather_issue(j):
            pltpu.async_copy(
                src.at[vsi.at[lax.rem(j, 3 * IDXB)]],
                vbuf.at[pl.ds(lax.rem(j, NBUF) * 128, 128)], gsem)

        def gather_wait(j):
            pltpu.make_async_copy(
                src.at[vsi.at[lax.rem(j, 3 * IDXB)]],
                vbuf.at[pl.ds(lax.rem(j, NBUF) * 128, 128)], gsem).wait()

        def scatter_issue(j):
            pltpu.async_copy(
                vbuf.at[pl.ds(lax.rem(j, NBUF) * 128, 128)],
                acc.at[vdi.at[lax.rem(j, 3 * IDXB)]], ssem, add=True)

        def scatter_wait(j):
            pltpu.make_async_copy(
                vbuf.at[pl.ds(lax.rem(j, NBUF) * 128, 128)],
                acc.at[vdi.at[lax.rem(j, 3 * IDXB)]], ssem).wait()

        pltpu.sync_copy(zeros.at[pl.ds(0, zrows)],
                        acc.at[pl.ds(s * zrows, zrows)])
        idx_issue(0, 0)
        idx_issue(1, 1)
        idx_wait()
        plsc.subcore_barrier()

        gather_issue(0)
        gather_issue(1)
        gather_issue(2)

        def body(j, carry):
            gather_wait(j)
            scatter_issue(j)

            @pl.when(j >= 1)
            def _():
                scatter_wait(j - 1)

            @pl.when(lax.rem(j, IDXB) == 0)
            def _():
                b = j // IDXB

                @pl.when(b + 1 <= NBLK - 1)
                def _():
                    idx_wait()

                @pl.when(b + 2 <= NBLK - 1)
                def _():
                    idx_issue(b + 2, lax.rem(b + 2, 3))

            @pl.when(j + 3 <= RPT - 1)
            def _():
                gather_issue(j + 3)
            return carry

        lax.fori_loop(0, RPT, body, 0)
        scatter_wait(RPT - 1)
        plsc.subcore_barrier()

        pltpu.sync_copy(acc.at[pl.ds(s * orows, orows)],
                        out.at[pl.ds(c * n_dst_pad + s * orows, orows)])

    return spmm


_spmm_u2i = _make_spmm(I_PAD, I_ACC)
_spmm_i2u = _make_spmm(U_PAD, U_ACC)


@functools.partial(
    pl.kernel,
    out_type=jax.ShapeDtypeStruct((2 * DEG_PAD, 1), jnp.float32),
    mesh=_MESH,
    scratch_types=[
        pltpu.VMEM((RPT, 128), jnp.int32),
        pltpu.VMEM((128, 1), jnp.float32),
        pltpu.VMEM_SHARED((DEG_ACC, 1), jnp.float32),
        pltpu.SemaphoreType.DMA,
    ],
    compiler_params=pltpu.CompilerParams(use_tc_tiling_on_sc=False),
)
def _degrees(didx, zeros1, ones, out, vdi, ones_v, acc, sem):
    c = lax.axis_index("c")
    s = lax.axis_index("s")
    t0 = s * RPT
    zrows = DEG_ACC // N_TILES
    orows = DEG_PAD // N_TILES

    pltpu.sync_copy(ones, ones_v)
    pltpu.sync_copy(zeros1.at[pl.ds(0, zrows)],
                    acc.at[pl.ds(s * zrows, zrows)])
    pltpu.sync_copy(didx.at[pl.ds(c * ROWS + t0, RPT)], vdi)
    plsc.subcore_barrier()

    def chunk_body(k, carry):
        base = k * CHUNK
        descs = []
        for j in range(CHUNK):
            descs.append(
                pltpu.async_copy(ones_v, acc.at[vdi.at[base + j]], sem,
                                 add=True))
        for d in descs:
            d.wait()
        return carry

    lax.fori_loop(0, N_CHUNKS, chunk_body, 0)
    plsc.subcore_barrier()
    pltpu.sync_copy(acc.at[pl.ds(s * orows, orows)],
                    out.at[pl.ds(c * DEG_PAD + s * orows, orows)])


def _half(x, n, n_pad):
    q = x.reshape(n, 2, HALF).transpose(1, 0, 2)
    q = jnp.pad(q, ((0, 0), (0, n_pad - n), (0, 0)))
    return q.reshape(2 * n_pad, HALF)


def _unhalf(x, n, n_pad):
    return x.reshape(2, n_pad, HALF)[:, :n].transpose(1, 0, 2).reshape(n, 64)


def _mult(vals, n, n_pad):
    return jnp.tile(jnp.pad(vals, (0, n_pad - n)), 2)[:, None]


def kernel(user_emb, item_emb, edge_row, edge_col):
    pad = EP - E_N
    pr = jnp.concatenate([edge_row, jnp.full((pad,), U_N, jnp.int32)])
    pc = jnp.concatenate([edge_col, jnp.full((pad,), I_N, jnp.int32)])
    dsti = jnp.concatenate([pr, pc]).reshape(2 * ROWS, 128)
    row_dst = dsti[:ROWS]
    col_dst = dsti[ROWS:]

    prs = jnp.concatenate([edge_row, jnp.zeros((pad,), jnp.int32)])
    pcs = jnp.concatenate([edge_col, jnp.zeros((pad,), jnp.int32)])
    usrc = jnp.concatenate([prs, prs + U_PAD]).reshape(2 * ROWS, 128)
    isrc = jnp.concatenate([pcs, pcs + I_PAD]).reshape(2 * ROWS, 128)

    zeros_sp = jnp.zeros((U_ACC // N_TILES, HALF), jnp.float32)
    zeros_1 = jnp.zeros((DEG_ACC // N_TILES, 1), jnp.float32)
    ones_128 = jnp.ones((128, 1), jnp.float32)

    deg = _degrees(dsti, zeros_1, ones_128)[:, 0]
    u_deg = jnp.where(deg[:U_N] == 0, 1.0, deg[:U_N])
    i_deg_raw = deg[DEG_PAD:DEG_PAD + I_N]
    i_deg = jnp.where(i_deg_raw == 0, 1.0, i_deg_raw)
    m_dus = _mult(u_deg ** -0.5, U_N, U_PAD)
    m_dis = _mult(i_deg ** -0.5, I_N, I_PAD)
    m_dui = _mult(1.0 / u_deg, U_N, U_PAD)
    m_dii = _mult(1.0 / i_deg, I_N, I_PAD)

    def u2i(x):
        return _spmm_u2i(x, usrc, col_dst, zeros_sp)

    def i2u(x):
        return _spmm_i2u(x, isrc, row_dst, zeros_sp)

    ue = _half(user_emb, U_N, U_PAD)
    ie = _half(item_emb, I_N, I_PAD)
    sum_u = ue
    sum_i = ie

    for _ in range(2):
        t = u2i(ue * m_dus)
        t = i2u(t * m_dii)
        ue = t * m_dus
        sum_u = sum_u + ue
        t = i2u(ie * m_dis)
        t = u2i(t * m_dui)
        ie = t * m_dis
        sum_i = sum_i + ie

    user_out = _unhalf(sum_u / 3.0, U_N, U_PAD)
    item_out = _unhalf(sum_i / 3.0, I_N, I_PAD)
    return (user_out, item_out)
